# Initial kernel scaffold; baseline (speedup 1.0000x reference)
import jax
import jax.numpy as jnp
from jax.experimental import pallas as pl


def kernel(iid, edge_index, edge_weight, emb, W1, W2, W_ih, W_hh, b_ih, b_hh):
    raise NotImplementedError("write your pallas kernel here")


# speedup vs baseline: 2.9003x; 2.9003x over previous
import functools

import jax
import jax.numpy as jnp
from jax import lax
from jax.experimental import pallas as pl
from jax.experimental.pallas import tpu as pltpu
from jax.experimental.pallas import tpu_sc as plsc

NC, NS, LANES = 2, 16, 16
NW = NC * NS
N_PAD = 10240
D = 128
E = 320000
EPT = E // NS
CH = 80
NCH = EPT // CH

def _get_mesh():
    return plsc.VectorSubcoreMesh(
        core_axis_name="c", subcore_axis_name="s",
        num_cores=NC, num_subcores=NS)


def _rsqrt_nr(s):
    i = lax.bitcast_convert_type(s, jnp.int32)
    i = jnp.int32(0x5F3759DF) - lax.shift_right_logical(i, 1)
    y = lax.bitcast_convert_type(i, jnp.float32)
    for _ in range(3):
        y = y * (1.5 - 0.5 * s * y * y)
    return y


def _gather_norm_body(iid_hbm, emb_hbm, out_hbm, idxb, rows, sem):
    c = lax.axis_index("c")
    sid = lax.axis_index("s")
    wid = sid * NC + c
    base = wid * (N_PAD // NW)

    def do_chunk(k, carry):
        off = base + k * 64
        pltpu.sync_copy(iid_hbm.at[pl.ds(off, 64)], idxb)
        pltpu.async_copy(emb_hbm.at[idxb], rows, sem).wait()

        def row_body(e, cc):
            acc = jnp.zeros((LANES,), jnp.float32)
            for j in range(8):
                v = rows[e, pl.ds(16 * j, 16)]
                acc = acc + v * v
            vals = [acc[i] for i in range(LANES)]
            while len(vals) > 1:
                vals = [vals[i] + vals[i + 1] for i in range(0, len(vals), 2)]
            ssq = jnp.maximum(vals[0], 1e-30)
            inv = _rsqrt_nr(ssq)
            for j in range(8):
                rows[e, pl.ds(16 * j, 16)] = rows[e, pl.ds(16 * j, 16)] * inv
            return cc

        lax.fori_loop(0, 64, row_body, 0)
        pltpu.sync_copy(rows, out_hbm.at[pl.ds(off, 64)])
        return carry

    lax.fori_loop(0, (N_PAD // NW) // 64, do_chunk, 0)


def _gather_normalize(iid_pad, emb):
    return pl.kernel(
        _gather_norm_body,
        out_type=jax.ShapeDtypeStruct((N_PAD, D), jnp.float32),
        mesh=_get_mesh(),
        scratch_types=[
            pltpu.VMEM((64,), jnp.int32),
            pltpu.VMEM((64, D), jnp.float32),
            pltpu.SemaphoreType.DMA,
        ],
    )(iid_pad, emb)


def _num_body(src_hbm, dst_hbm, w_hbm, feat_hbm, z128_hbm, num_out,
              sidx, didx, wv, gbuf, sbuf, accum, sem):
    c = lax.axis_index("c")
    sid = lax.axis_index("s")

    pltpu.sync_copy(z128_hbm, accum.at[pl.ds(sid * 640, 640)])
    plsc.subcore_barrier()

    tb = sid * EPT

    def direction(g_hbm, s_hbm):
        def chunk(i, carry):
            off = tb + i * CH
            pltpu.sync_copy(g_hbm.at[pl.ds(off, CH)], sidx)
            pltpu.sync_copy(s_hbm.at[pl.ds(off, CH)], didx)
            pltpu.sync_copy(w_hbm.at[pl.ds(off, CH)], wv)
            pltpu.async_copy(feat_hbm.at[sidx], gbuf, sem).wait()

            for g in range(CH // 16):
                wg = wv[pl.ds(g * 16, 16)]
                for j in range(LANES):
                    e = g * 16 + j
                    we = wg[j]
                    for jj in range(8):
                        sbuf[e, pl.ds(16 * jj, 16)] = (
                            gbuf[e, pl.ds(16 * jj, 16)] * we)

            pltpu.sync_copy(sbuf, accum.at[didx], add=True)
            return carry

        lax.fori_loop(0, NCH, chunk, 0)

    @pl.when(c == 0)
    def _():
        direction(src_hbm, dst_hbm)

    @pl.when(c == 1)
    def _():
        direction(dst_hbm, src_hbm)

    plsc.subcore_barrier()
    rb = sid * 640
    pltpu.sync_copy(accum.at[pl.ds(rb, 640)], num_out.at[c, pl.ds(rb, 640)])


def _aggregate_num(src, dst, w, feat_pad):
    return pl.kernel(
        _num_body,
        out_type=jax.ShapeDtypeStruct((NC, N_PAD, D), jnp.float32),
        mesh=_get_mesh(),
        scratch_types=[
            pltpu.VMEM((CH,), jnp.int32),
            pltpu.VMEM((CH,), jnp.int32),
            pltpu.VMEM((CH,), jnp.float32),
            pltpu.VMEM((CH, D), jnp.float32),
            pltpu.VMEM((CH, D), jnp.float32),
            pltpu.VMEM_SHARED((N_PAD, D), jnp.float32),
            pltpu.SemaphoreType.DMA,
        ],
    )(src, dst, w, feat_pad, jnp.zeros((640, D), jnp.float32))


def _den_body(src_hbm, dst_hbm, w_hbm, z128_hbm, den_out,
              didx, wv, sden, dacc, sem):
    c = lax.axis_index("c")
    sid = lax.axis_index("s")
    lane_iota = lax.iota(jnp.int32, LANES)

    def zrow(e, cc):
        zv = jnp.zeros((LANES,), jnp.float32)
        for jj in range(1, 8):
            sden[e, pl.ds(16 * jj, 16)] = zv
        return cc

    lax.fori_loop(0, CH, zrow, 0)
    pltpu.sync_copy(z128_hbm, dacc.at[pl.ds(sid * 640, 640)])
    plsc.subcore_barrier()

    tb = sid * EPT

    def direction(s_hbm):
        def chunk(i, carry):
            off = tb + i * CH
            pltpu.sync_copy(s_hbm.at[pl.ds(off, CH)], didx)
            pltpu.sync_copy(w_hbm.at[pl.ds(off, CH)], wv)

            for g in range(CH // 16):
                wg = wv[pl.ds(g * 16, 16)]
                dg = didx[pl.ds(g * 16, 16)]
                for j in range(LANES):
                    e = g * 16 + j
                    sden[e, pl.ds(0, 16)] = jnp.where(
                        lane_iota == (dg[j] & (LANES - 1)), wg[j], 0.0)

            pltpu.sync_copy(sden, dacc.at[didx], add=True)
            return carry

        lax.fori_loop(0, NCH, chunk, 0)

    @pl.when(c == 0)
    def _():
        direction(dst_hbm)

    @pl.when(c == 1)
    def _():
        direction(src_hbm)

    plsc.subcore_barrier()
    rb = sid * 640
    pltpu.sync_copy(dacc.at[pl.ds(rb, 640)], den_out.at[c, pl.ds(rb, 640)])


def _aggregate_den(src, dst, w):
    return pl.kernel(
        _den_body,
        out_type=jax.ShapeDtypeStruct((NC, N_PAD, D), jnp.float32),
        mesh=_get_mesh(),
        scratch_types=[
            pltpu.VMEM((CH,), jnp.int32),
            pltpu.VMEM((CH,), jnp.float32),
            pltpu.VMEM((CH, D), jnp.float32),
            pltpu.VMEM_SHARED((N_PAD, D), jnp.float32),
            pltpu.SemaphoreType.DMA,
        ],
    )(src, dst, w, jnp.zeros((640, D), jnp.float32))


def _dense_body(num_ref, den_ref, feat_ref, w1t, w2t, wiht, whht, bih, bhh,
                out_ref):
    dot = functools.partial(jnp.dot, precision=lax.Precision.HIGHEST,
                            preferred_element_type=jnp.float32)
    num1, num2 = num_ref[0], num_ref[1]
    den1 = jnp.sum(den_ref[0][:, :16], axis=-1, keepdims=True)
    den2 = jnp.sum(den_ref[1][:, :16], axis=-1, keepdims=True)
    neigh1 = jnp.where(den1 > 0, num1 / jnp.maximum(den1, 1e-12), 0.0)
    neigh2 = jnp.where(den2 > 0, num2 / jnp.maximum(den2, 1e-12), 0.0)
    f = feat_ref[...]
    n1 = dot(neigh1, w1t[...])
    n2 = dot(neigh2, w2t[...])
    wih = wiht[...]
    gi = dot(n1, wih[:256, :]) + dot(n2, wih[256:, :]) + bih[...]
    gh = dot(f, whht[...]) + bhh[...]
    r = jax.nn.sigmoid(gi[:, :D] + gh[:, :D])
    z = jax.nn.sigmoid(gi[:, D:2 * D] + gh[:, D:2 * D])
    cand = jnp.tanh(gi[:, 2 * D:] + r * gh[:, 2 * D:])
    out_ref[...] = (1.0 - z) * cand + z * f


def _dense(num2, den3, feat_pad, W1, W2, W_ih, W_hh, b_ih, b_hh):
    R = 2048
    grid = (N_PAD // R,)
    return pl.pallas_call(
        _dense_body,
        grid=grid,
        in_specs=[
            pl.BlockSpec((NC, R, D), lambda i: (0, i, 0)),
            pl.BlockSpec((NC, R, D), lambda i: (0, i, 0)),
            pl.BlockSpec((R, D), lambda i: (i, 0)),
            pl.BlockSpec((D, 256), lambda i: (0, 0)),
            pl.BlockSpec((D, 256), lambda i: (0, 0)),
            pl.BlockSpec((512, 384), lambda i: (0, 0)),
            pl.BlockSpec((D, 384), lambda i: (0, 0)),
            pl.BlockSpec((1, 384), lambda i: (0, 0)),
            pl.BlockSpec((1, 384), lambda i: (0, 0)),
        ],
        out_specs=pl.BlockSpec((R, D), lambda i: (i, 0)),
        out_shape=jax.ShapeDtypeStruct((N_PAD, D), jnp.float32),
    )(num2, den3, feat_pad, W1.T, W2.T, W_ih.T, W_hh.T,
      b_ih[None, :], b_hh[None, :])


def kernel(iid, edge_index, edge_weight, emb, W1, W2, W_ih, W_hh, b_ih, b_hh):
    n = iid.shape[0]
    iid32 = iid.astype(jnp.int32)
    iid_pad = jnp.concatenate(
        [iid32, jnp.zeros((N_PAD - n,), jnp.int32)])
    src = edge_index[0].astype(jnp.int32)
    dst = edge_index[1].astype(jnp.int32)
    w = edge_weight.astype(jnp.float32)

    feat_pad = _gather_normalize(iid_pad, emb)
    num2 = _aggregate_num(src, dst, w, feat_pad)
    den3 = _aggregate_den(src, dst, w)
    h_pad = _dense(num2, den3, feat_pad, W1, W2, W_ih, W_hh, b_ih, b_hh)
    return h_pad[:n]


# speedup vs baseline: 3.1436x; 1.0839x over previous
import functools

import jax
import jax.numpy as jnp
from jax import lax
from jax.experimental import pallas as pl
from jax.experimental.pallas import tpu as pltpu
from jax.experimental.pallas import tpu_sc as plsc

NC, NS, LANES = 2, 16, 16
NW = NC * NS
N_PAD = 10240
D = 128
E = 320000
EPT = E // NS
CH = 80
NCH = EPT // CH
CH2 = 128
EPT2 = 20480
NCH2 = EPT2 // CH2
E_PAD = NS * EPT2 + 2 * CH2

def _get_mesh():
    return plsc.VectorSubcoreMesh(
        core_axis_name="c", subcore_axis_name="s",
        num_cores=NC, num_subcores=NS)


def _rsqrt_nr(s):
    i = lax.bitcast_convert_type(s, jnp.int32)
    i = jnp.int32(0x5F3759DF) - lax.shift_right_logical(i, 1)
    y = lax.bitcast_convert_type(i, jnp.float32)
    for _ in range(3):
        y = y * (1.5 - 0.5 * s * y * y)
    return y


def _gather_norm_body(iid_hbm, emb_hbm, out_hbm, idxb, rows, sem):
    c = lax.axis_index("c")
    sid = lax.axis_index("s")
    wid = sid * NC + c
    base = wid * (N_PAD // NW)

    def do_chunk(k, carry):
        off = base + k * 64
        pltpu.sync_copy(iid_hbm.at[pl.ds(off, 64)], idxb)
        pltpu.async_copy(emb_hbm.at[idxb], rows, sem).wait()

        def row_body(e, cc):
            acc = jnp.zeros((LANES,), jnp.float32)
            for j in range(8):
                v = rows[e, pl.ds(16 * j, 16)]
                acc = acc + v * v
            vals = [acc[i] for i in range(LANES)]
            while len(vals) > 1:
                vals = [vals[i] + vals[i + 1] for i in range(0, len(vals), 2)]
            ssq = jnp.maximum(vals[0], 1e-30)
            inv = _rsqrt_nr(ssq)
            for j in range(8):
                rows[e, pl.ds(16 * j, 16)] = rows[e, pl.ds(16 * j, 16)] * inv
            return cc

        lax.fori_loop(0, 64, row_body, 0)
        pltpu.sync_copy(rows, out_hbm.at[pl.ds(off, 64)])
        return carry

    lax.fori_loop(0, (N_PAD // NW) // 64, do_chunk, 0)


def _gather_normalize(iid_pad, emb):
    return pl.kernel(
        _gather_norm_body,
        out_type=jax.ShapeDtypeStruct((N_PAD, D), jnp.float32),
        mesh=_get_mesh(),
        scratch_types=[
            pltpu.VMEM((64,), jnp.int32),
            pltpu.VMEM((64, D), jnp.float32),
            pltpu.SemaphoreType.DMA,
        ],
    )(iid_pad, emb)


def _num_body(src_hbm, dst_hbm, w_hbm, feat_hbm, z128_hbm, num_out,
              sidx0, didx0, wv0, gbuf0,
              sidx1, didx1, wv1, gbuf1,
              accum, semg0, semg1, semi0, semi1):
    c = lax.axis_index("c")
    sid = lax.axis_index("s")

    pltpu.sync_copy(z128_hbm, accum.at[pl.ds(sid * 640, 640)])
    plsc.subcore_barrier()

    tb = sid * EPT2
    S = ((sidx0, didx0, wv0, gbuf0, semg0, semi0),
         (sidx1, didx1, wv1, gbuf1, semg1, semi1))

    def direction(g_hbm, s_hbm):
        def load_idx(bs, j, sync):
            off = tb + j * CH2
            if sync:
                pltpu.sync_copy(g_hbm.at[pl.ds(off, CH2)], bs[0])
                pltpu.sync_copy(s_hbm.at[pl.ds(off, CH2)], bs[1])
                pltpu.sync_copy(w_hbm.at[pl.ds(off, CH2)], bs[2])
            else:
                pltpu.async_copy(g_hbm.at[pl.ds(off, CH2)], bs[0], bs[5])
                pltpu.async_copy(s_hbm.at[pl.ds(off, CH2)], bs[1], bs[5])
                pltpu.async_copy(w_hbm.at[pl.ds(off, CH2)], bs[2], bs[5])

        def wait_idx(bs):
            pltpu.make_async_copy(g_hbm.at[pl.ds(tb, CH2)], bs[0], bs[5]).wait()
            pltpu.make_async_copy(s_hbm.at[pl.ds(tb, CH2)], bs[1], bs[5]).wait()
            pltpu.make_async_copy(w_hbm.at[pl.ds(tb, CH2)], bs[2], bs[5]).wait()

        def start_gather(bs):
            pltpu.async_copy(feat_hbm.at[bs[0]], bs[3], bs[4])

        def wait_gather(bs):
            pltpu.make_async_copy(feat_hbm.at[bs[0]], bs[3], bs[4]).wait()

        def scale_scatter(bs):
            didx, wv, gbuf = bs[1], bs[2], bs[3]
            for g in range(CH2 // 16):
                wg = wv[pl.ds(g * 16, 16)]
                for j in range(LANES):
                    e = g * 16 + j
                    we = wg[j]
                    for jj in range(8):
                        gbuf[e, pl.ds(16 * jj, 16)] = (
                            gbuf[e, pl.ds(16 * jj, 16)] * we)
            pltpu.sync_copy(gbuf, accum.at[didx], add=True)

        load_idx(S[0], 0, True)
        start_gather(S[0])
        load_idx(S[1], 1, False)

        @pl.loop(0, NCH2, step=2)
        def _(g0):
            for b in range(2):
                j = g0 + b
                q = 1 - b
                wait_idx(S[q])
                start_gather(S[q])
                wait_gather(S[b])
                scale_scatter(S[b])
                load_idx(S[b], j + 2, False)

        wait_gather(S[0])
        wait_idx(S[1])

    @pl.when(c == 0)
    def _():
        direction(src_hbm, dst_hbm)

    @pl.when(c == 1)
    def _():
        direction(dst_hbm, src_hbm)

    plsc.subcore_barrier()
    rb = sid * 640
    pltpu.sync_copy(accum.at[pl.ds(rb, 640)], num_out.at[c, pl.ds(rb, 640)])


def _aggregate_num(src, dst, w, feat_pad):
    return pl.kernel(
        _num_body,
        out_type=jax.ShapeDtypeStruct((NC, N_PAD, D), jnp.float32),
        mesh=_get_mesh(),
        scratch_types=[
            pltpu.VMEM((CH2,), jnp.int32),
            pltpu.VMEM((CH2,), jnp.int32),
            pltpu.VMEM((CH2,), jnp.float32),
            pltpu.VMEM((CH2, D), jnp.float32),
            pltpu.VMEM((CH2,), jnp.int32),
            pltpu.VMEM((CH2,), jnp.int32),
            pltpu.VMEM((CH2,), jnp.float32),
            pltpu.VMEM((CH2, D), jnp.float32),
            pltpu.VMEM_SHARED((N_PAD, D), jnp.float32),
            pltpu.SemaphoreType.DMA,
            pltpu.SemaphoreType.DMA,
            pltpu.SemaphoreType.DMA,
            pltpu.SemaphoreType.DMA,
        ],
    )(src, dst, w, feat_pad, jnp.zeros((640, D), jnp.float32))


def _den_body(src_hbm, dst_hbm, w_hbm, z128_hbm, den_out,
              didx0, wv0, didx1, wv1, sden, dacc, semi0, semi1):
    c = lax.axis_index("c")
    sid = lax.axis_index("s")
    lane_iota = lax.iota(jnp.int32, LANES)

    def zrow(e, cc):
        zv = jnp.zeros((LANES,), jnp.float32)
        for jj in range(1, 8):
            sden[e, pl.ds(16 * jj, 16)] = zv
        return cc

    lax.fori_loop(0, CH2, zrow, 0)
    pltpu.sync_copy(z128_hbm, dacc.at[pl.ds(sid * 640, 640)])
    plsc.subcore_barrier()

    tb = sid * EPT2
    S = ((didx0, wv0, semi0), (didx1, wv1, semi1))

    def direction(s_hbm):
        def load_idx(bs, j):
            off = tb + j * CH2
            pltpu.async_copy(s_hbm.at[pl.ds(off, CH2)], bs[0], bs[2])
            pltpu.async_copy(w_hbm.at[pl.ds(off, CH2)], bs[1], bs[2])

        def wait_idx(bs):
            pltpu.make_async_copy(s_hbm.at[pl.ds(tb, CH2)], bs[0], bs[2]).wait()
            pltpu.make_async_copy(w_hbm.at[pl.ds(tb, CH2)], bs[1], bs[2]).wait()

        def build_scatter(bs):
            didx, wv = bs[0], bs[1]
            for g in range(CH2 // 16):
                wg = wv[pl.ds(g * 16, 16)]
                dg = didx[pl.ds(g * 16, 16)]
                for j in range(LANES):
                    e = g * 16 + j
                    sden[e, pl.ds(0, 16)] = jnp.where(
                        lane_iota == (dg[j] & (LANES - 1)), wg[j], 0.0)
            pltpu.sync_copy(sden, dacc.at[didx], add=True)

        load_idx(S[0], 0)
        load_idx(S[1], 1)

        @pl.loop(0, NCH2, step=2)
        def _(g0):
            for b in range(2):
                j = g0 + b
                wait_idx(S[b])
                build_scatter(S[b])
                load_idx(S[b], j + 2)

        wait_idx(S[0])
        wait_idx(S[1])

    @pl.when(c == 0)
    def _():
        direction(dst_hbm)

    @pl.when(c == 1)
    def _():
        direction(src_hbm)

    plsc.subcore_barrier()
    rb = sid * 640
    pltpu.sync_copy(dacc.at[pl.ds(rb, 640)], den_out.at[c, pl.ds(rb, 640)])


def _aggregate_den(src, dst, w):
    return pl.kernel(
        _den_body,
        out_type=jax.ShapeDtypeStruct((NC, N_PAD, D), jnp.float32),
        mesh=_get_mesh(),
        scratch_types=[
            pltpu.VMEM((CH2,), jnp.int32),
            pltpu.VMEM((CH2,), jnp.float32),
            pltpu.VMEM((CH2,), jnp.int32),
            pltpu.VMEM((CH2,), jnp.float32),
            pltpu.VMEM((CH2, D), jnp.float32),
            pltpu.VMEM_SHARED((N_PAD, D), jnp.float32),
            pltpu.SemaphoreType.DMA,
            pltpu.SemaphoreType.DMA,
        ],
    )(src, dst, w, jnp.zeros((640, D), jnp.float32))


def _dense_body(num_ref, den_ref, feat_ref, w1t, w2t, wiht, whht, bih, bhh,
                out_ref):
    dot = functools.partial(jnp.dot, precision=lax.Precision.HIGHEST,
                            preferred_element_type=jnp.float32)
    num1, num2 = num_ref[0], num_ref[1]
    den1 = jnp.sum(den_ref[0][:, :16], axis=-1, keepdims=True)
    den2 = jnp.sum(den_ref[1][:, :16], axis=-1, keepdims=True)
    neigh1 = jnp.where(den1 > 0, num1 / jnp.maximum(den1, 1e-12), 0.0)
    neigh2 = jnp.where(den2 > 0, num2 / jnp.maximum(den2, 1e-12), 0.0)
    f = feat_ref[...]
    n1 = dot(neigh1, w1t[...])
    n2 = dot(neigh2, w2t[...])
    wih = wiht[...]
    gi = dot(n1, wih[:256, :]) + dot(n2, wih[256:, :]) + bih[...]
    gh = dot(f, whht[...]) + bhh[...]
    r = jax.nn.sigmoid(gi[:, :D] + gh[:, :D])
    z = jax.nn.sigmoid(gi[:, D:2 * D] + gh[:, D:2 * D])
    cand = jnp.tanh(gi[:, 2 * D:] + r * gh[:, 2 * D:])
    out_ref[...] = (1.0 - z) * cand + z * f


def _dense(num2, den3, feat_pad, W1, W2, W_ih, W_hh, b_ih, b_hh):
    R = 2048
    grid = (N_PAD // R,)
    return pl.pallas_call(
        _dense_body,
        grid=grid,
        in_specs=[
            pl.BlockSpec((NC, R, D), lambda i: (0, i, 0)),
            pl.BlockSpec((NC, R, D), lambda i: (0, i, 0)),
            pl.BlockSpec((R, D), lambda i: (i, 0)),
            pl.BlockSpec((D, 256), lambda i: (0, 0)),
            pl.BlockSpec((D, 256), lambda i: (0, 0)),
            pl.BlockSpec((512, 384), lambda i: (0, 0)),
            pl.BlockSpec((D, 384), lambda i: (0, 0)),
            pl.BlockSpec((1, 384), lambda i: (0, 0)),
            pl.BlockSpec((1, 384), lambda i: (0, 0)),
        ],
        out_specs=pl.BlockSpec((R, D), lambda i: (i, 0)),
        out_shape=jax.ShapeDtypeStruct((N_PAD, D), jnp.float32),
    )(num2, den3, feat_pad, W1.T, W2.T, W_ih.T, W_hh.T,
      b_ih[None, :], b_hh[None, :])


def kernel(iid, edge_index, edge_weight, emb, W1, W2, W_ih, W_hh, b_ih, b_hh):
    n = iid.shape[0]
    iid32 = iid.astype(jnp.int32)
    iid_pad = jnp.concatenate(
        [iid32, jnp.zeros((N_PAD - n,), jnp.int32)])
    e_n = edge_index.shape[1]
    ezpad = jnp.zeros((E_PAD - e_n,), jnp.int32)
    src = jnp.concatenate([edge_index[0].astype(jnp.int32), ezpad])
    dst = jnp.concatenate([edge_index[1].astype(jnp.int32), ezpad])
    w = jnp.concatenate([edge_weight.astype(jnp.float32),
                         jnp.zeros((E_PAD - e_n,), jnp.float32)])

    feat_pad = _gather_normalize(iid_pad, emb)
    num2 = _aggregate_num(src, dst, w, feat_pad)
    den3 = _aggregate_den(src, dst, w)
    h_pad = _dense(num2, den3, feat_pad, W1, W2, W_ih, W_hh, b_ih, b_hh)
    return h_pad[:n]
